# Initial kernel scaffold; baseline (speedup 1.0000x reference)
import functools

import jax
import jax.numpy as jnp
from jax import lax
from jax.experimental import pallas as pl
from jax.experimental.pallas import tpu as pltpu
from jax.experimental.pallas import tpu_sc as plsc

B = 4096
INPUT_BITS = 1024
N_IN = 2048
N_STATE = 2048
N_OUT = 1024
NB = 12
ADDR = 1 << NB

NC = 2
NS = 16
NW = NC * NS


def _build_w_body(t_off, connt_ref, w_ref):
    tb, nb = w_ref.shape
    i = pl.program_id(0)
    t_iota = lax.broadcasted_iota(jnp.int32, (tb, nb), 0) + (i * tb + t_off)
    acc = jnp.zeros((tb, nb), jnp.float32)
    for k in range(NB):
        row = connt_ref[k, :][None, :]
        acc = acc + jnp.where(row == t_iota, jnp.float32(1 << k), 0.0)
    w_ref[...] = acc


def _build_w(connt, t_size, t_off):
    n = connt.shape[1]
    tb = min(512, t_size)
    nb = min(512, n)
    return pl.pallas_call(
        functools.partial(_build_w_body, t_off),
        grid=(t_size // tb, n // nb),
        in_specs=[pl.BlockSpec((16, nb), lambda i, j: (0, j))],
        out_specs=pl.BlockSpec((tb, nb), lambda i, j: (i, j)),
        out_shape=jax.ShapeDtypeStruct((t_size, n), jnp.float32),
    )(connt)


def _pack_body(mem_ref, p_ref):
    jrows, ab = p_ref.shape
    n = mem_ref.shape[0]
    j_iota = lax.broadcasted_iota(jnp.int32, (jrows, n), 0)
    n_iota = lax.broadcasted_iota(jnp.int32, (jrows, n), 1)
    inrow = (n_iota >> 5) == j_iota
    sh = n_iota & 31
    pw_lo = jnp.where(inrow & (sh < 16), (1 << (sh & 15)), 0).astype(jnp.float32)
    pw_hi = jnp.where(inrow & (sh >= 16), (1 << (sh & 15)), 0).astype(jnp.float32)
    memf = mem_ref[...].astype(jnp.float32)
    dn = (((1,), (0,)), ((), ()))
    lo = lax.dot_general(pw_lo, memf, dn, preferred_element_type=jnp.float32)
    hi = lax.dot_general(pw_hi, memf, dn, preferred_element_type=jnp.float32)
    lo_i = (lo + 0.5).astype(jnp.int32)
    hi_i = (hi + 0.5).astype(jnp.int32)
    p_ref[...] = lo_i | (hi_i << 16)


def _pack(mem):
    n, a = mem.shape
    jrows = n // 32
    ab = 512
    return pl.pallas_call(
        _pack_body,
        grid=(a // ab,),
        in_specs=[pl.BlockSpec((n, ab), lambda j: (0, j))],
        out_specs=pl.BlockSpec((jrows, ab), lambda j: (0, j)),
        out_shape=jax.ShapeDtypeStruct((jrows, a), jnp.int32),
    )(mem)


_DN_C00 = (((0,), (0,)), ((), ()))


def _mm_body(w_ref, x_ref, o_ref):
    acc = lax.dot_general(w_ref[...], x_ref[...], _DN_C00,
                          preferred_element_type=jnp.float32)
    o_ref[...] = (acc + 0.5).astype(jnp.int32)


def _mm(w, x, bb=512):
    t, n = w.shape
    b = x.shape[1]
    return pl.pallas_call(
        _mm_body,
        grid=(b // bb,),
        in_specs=[pl.BlockSpec((t, n), lambda j: (0, 0)),
                  pl.BlockSpec((t, bb), lambda j: (0, j))],
        out_specs=pl.BlockSpec((n, bb), lambda j: (0, j)),
        out_shape=jax.ShapeDtypeStruct((n, b), jnp.int32),
    )(w, x)


def _mm2_body(wa_ref, xa_ref, wb_ref, xb_ref, o_ref):
    acc = lax.dot_general(wa_ref[...], xa_ref[...], _DN_C00,
                          preferred_element_type=jnp.float32)
    acc = acc + lax.dot_general(wb_ref[...], xb_ref[...], _DN_C00,
                                preferred_element_type=jnp.float32)
    o_ref[...] = (acc + 0.5).astype(jnp.int32)


def _mm2(wa, xa, wb, xb, bb=512):
    t, n = wa.shape
    b = xa.shape[1]
    return pl.pallas_call(
        _mm2_body,
        grid=(b // bb,),
        in_specs=[pl.BlockSpec((t, n), lambda j: (0, 0)),
                  pl.BlockSpec((t, bb), lambda j: (0, j)),
                  pl.BlockSpec((t, n), lambda j: (0, 0)),
                  pl.BlockSpec((t, bb), lambda j: (0, j))],
        out_specs=pl.BlockSpec((n, bb), lambda j: (0, j)),
        out_shape=jax.ShapeDtypeStruct((n, b), jnp.int32),
    )(wa, xa, wb, xb)


_BC = 512


def _sc_gather(addrt, packed):
    n, b = addrt.shape
    npw = n // NW
    jrows = npw // 32
    nchunks = b // _BC
    mesh = plsc.VectorSubcoreMesh(core_axis_name="c", subcore_axis_name="s")

    @functools.partial(
        pl.kernel,
        mesh=mesh,
        out_type=jax.ShapeDtypeStruct((n, b), jnp.float32),
        scratch_types=[
            pltpu.VMEM((jrows, ADDR), jnp.int32),
            pltpu.VMEM((npw, _BC), jnp.int32),
            pltpu.VMEM((npw, _BC), jnp.float32),
        ],
    )
    def k(addr_hbm, p_hbm, out_hbm, table_v, addr_v, out_v):
        wid = lax.axis_index("s") * NC + lax.axis_index("c")
        n0 = wid * npw
        pltpu.sync_copy(p_hbm.at[pl.ds(wid * jrows, jrows)], table_v)

        def row_body(j, carry):
            wrow = jnp.broadcast_to(j >> 5, (16,)).astype(jnp.int32)
            sh = jnp.broadcast_to(j & 31, (16,)).astype(jnp.int32)
            for i in range(_BC // 16):
                a = addr_v[j, pl.ds(i * 16, 16)]
                w16 = plsc.load_gather(table_v, [wrow, a])
                bit = lax.shift_right_logical(w16, sh) & 1
                out_v[j, pl.ds(i * 16, 16)] = bit.astype(jnp.float32)
            return carry

        for c in range(nchunks):
            pltpu.sync_copy(
                addr_hbm.at[pl.ds(n0, npw), pl.ds(c * _BC, _BC)], addr_v)
            lax.fori_loop(0, npw, row_body, 0)
            pltpu.sync_copy(
                out_v, out_hbm.at[pl.ds(n0, npw), pl.ds(c * _BC, _BC)])

    return k(addrt, packed)


def kernel(input, conn_in, conn_state, conn_out, mem_in, mem_state, mem_out):
    pad = ((0, 16 - NB), (0, 0))
    connt_in = jnp.pad(conn_in.T, pad)
    connt_state = jnp.pad(conn_state.T, pad)
    connt_out = jnp.pad(conn_out.T, pad)

    w1 = _build_w(connt_in, INPUT_BITS, 0)
    w2 = _build_w(connt_state, N_IN, 0)
    w3a = _build_w(connt_out, N_IN, 0)
    w3b = _build_w(connt_out, N_STATE, N_IN)

    p1 = _pack(mem_in)
    p2 = _pack(mem_state)
    p3 = _pack(mem_out)

    x0t = input.astype(jnp.float32).T

    addr1t = _mm(w1, x0t)
    b1t = _sc_gather(addr1t, p1)
    addr2t = _mm(w2, b1t)
    b2t = _sc_gather(addr2t, p2)
    addr3t = _mm2(w3a, b1t, w3b, b2t)
    o3t = _sc_gather(addr3t, p3)

    return o3t.T.astype(bool)


# speedup vs baseline: 3.2261x; 3.2261x over previous
import functools

import jax
import jax.numpy as jnp
from jax import lax
from jax.experimental import pallas as pl
from jax.experimental.pallas import tpu as pltpu
from jax.experimental.pallas import tpu_sc as plsc

B = 4096
INPUT_BITS = 1024
N_IN = 2048
N_STATE = 2048
N_OUT = 1024
NB = 12
ADDR = 1 << NB

NC = 2
NS = 16
NW = NC * NS


def _build_w_body(t_off, connt_ref, w_ref):
    tb, nb = w_ref.shape
    i = pl.program_id(0)
    t_iota = lax.broadcasted_iota(jnp.int32, (tb, nb), 0) + (i * tb + t_off)
    acc = jnp.zeros((tb, nb), jnp.float32)
    for k in range(NB):
        row = connt_ref[k, :][None, :]
        acc = acc + jnp.where(row == t_iota, jnp.float32(1 << k), 0.0)
    w_ref[...] = acc


def _build_w(connt, t_size, t_off):
    n = connt.shape[1]
    tb = min(512, t_size)
    nb = min(512, n)
    return pl.pallas_call(
        functools.partial(_build_w_body, t_off),
        grid=(t_size // tb, n // nb),
        in_specs=[pl.BlockSpec((16, nb), lambda i, j: (0, j))],
        out_specs=pl.BlockSpec((tb, nb), lambda i, j: (i, j)),
        out_shape=jax.ShapeDtypeStruct((t_size, n), jnp.float32),
    )(connt)


def _pack_body(mem_ref, p_ref):
    jrows, ab = p_ref.shape
    n = mem_ref.shape[0]
    j_iota = lax.broadcasted_iota(jnp.int32, (jrows, n), 0)
    n_iota = lax.broadcasted_iota(jnp.int32, (jrows, n), 1)
    inrow = (n_iota >> 5) == j_iota
    sh = n_iota & 31
    pw_lo = jnp.where(inrow & (sh < 16), (1 << (sh & 15)), 0).astype(jnp.float32)
    pw_hi = jnp.where(inrow & (sh >= 16), (1 << (sh & 15)), 0).astype(jnp.float32)
    memf = mem_ref[...].astype(jnp.float32)
    dn = (((1,), (0,)), ((), ()))
    lo = lax.dot_general(pw_lo, memf, dn, preferred_element_type=jnp.float32, precision=lax.Precision.HIGHEST)
    hi = lax.dot_general(pw_hi, memf, dn, preferred_element_type=jnp.float32, precision=lax.Precision.HIGHEST)
    lo_i = (lo + 0.5).astype(jnp.int32)
    hi_i = (hi + 0.5).astype(jnp.int32)
    p_ref[...] = lo_i | (hi_i << 16)


def _pack(mem):
    n, a = mem.shape
    jrows = n // 32
    ab = 512
    return pl.pallas_call(
        _pack_body,
        grid=(a // ab,),
        in_specs=[pl.BlockSpec((n, ab), lambda j: (0, j))],
        out_specs=pl.BlockSpec((jrows, ab), lambda j: (0, j)),
        out_shape=jax.ShapeDtypeStruct((jrows, a), jnp.int32),
    )(mem)


_DN_C00 = (((0,), (0,)), ((), ()))


def _mm_body(w_ref, x_ref, o_ref):
    acc = lax.dot_general(w_ref[...], x_ref[...], _DN_C00,
                          preferred_element_type=jnp.float32, precision=lax.Precision.HIGHEST)
    o_ref[...] = (acc + 0.5).astype(jnp.int32)


def _mm(w, x, bb=512):
    t, n = w.shape
    b = x.shape[1]
    return pl.pallas_call(
        _mm_body,
        grid=(b // bb,),
        in_specs=[pl.BlockSpec((t, n), lambda j: (0, 0)),
                  pl.BlockSpec((t, bb), lambda j: (0, j))],
        out_specs=pl.BlockSpec((n, bb), lambda j: (0, j)),
        out_shape=jax.ShapeDtypeStruct((n, b), jnp.int32),
    )(w, x)


def _mm2_body(wa_ref, xa_ref, wb_ref, xb_ref, o_ref):
    acc = lax.dot_general(wa_ref[...], xa_ref[...], _DN_C00,
                          preferred_element_type=jnp.float32, precision=lax.Precision.HIGHEST)
    acc = acc + lax.dot_general(wb_ref[...], xb_ref[...], _DN_C00,
                                preferred_element_type=jnp.float32, precision=lax.Precision.HIGHEST)
    o_ref[...] = (acc + 0.5).astype(jnp.int32)


def _mm2(wa, xa, wb, xb, bb=512):
    t, n = wa.shape
    b = xa.shape[1]
    return pl.pallas_call(
        _mm2_body,
        grid=(b // bb,),
        in_specs=[pl.BlockSpec((t, n), lambda j: (0, 0)),
                  pl.BlockSpec((t, bb), lambda j: (0, j)),
                  pl.BlockSpec((t, n), lambda j: (0, 0)),
                  pl.BlockSpec((t, bb), lambda j: (0, j))],
        out_specs=pl.BlockSpec((n, bb), lambda j: (0, j)),
        out_shape=jax.ShapeDtypeStruct((n, b), jnp.int32),
    )(wa, xa, wb, xb)


_BC = 512


def _sc_gather(addrt, packed):
    n, b = addrt.shape
    npw = n // NW
    jrows = npw // 32
    nchunks = b // _BC
    mesh = plsc.VectorSubcoreMesh(core_axis_name="c", subcore_axis_name="s",
                                  num_cores=NC)

    @functools.partial(
        pl.kernel,
        mesh=mesh,
        compiler_params=pltpu.CompilerParams(needs_layout_passes=False),
        out_type=jax.ShapeDtypeStruct((n, b), jnp.float32),
        scratch_types=[
            pltpu.VMEM((jrows * ADDR,), jnp.int32),
            pltpu.VMEM((npw, _BC), jnp.int32),
            pltpu.VMEM((npw, _BC), jnp.float32),
        ],
    )
    def k(addr_hbm, p_hbm, out_hbm, table_v, addr_v, out_v):
        wid = lax.axis_index("s") * NC + lax.axis_index("c")
        n0 = wid * npw
        pltpu.sync_copy(p_hbm.at[pl.ds(wid * jrows * ADDR, jrows * ADDR)],
                        table_v)

        def row_body(j, carry):
            base = jnp.broadcast_to((j >> 5) * ADDR, (16,)).astype(jnp.int32)
            sh = jnp.broadcast_to(j & 31, (16,)).astype(jnp.int32)
            for i in range(_BC // 16):
                a = addr_v[j, pl.ds(i * 16, 16)]
                w16 = plsc.load_gather(table_v, [base + a])
                bit = lax.shift_right_logical(w16, sh) & 1
                out_v[j, pl.ds(i * 16, 16)] = bit.astype(jnp.float32)
            return carry

        for c in range(nchunks):
            pltpu.sync_copy(
                addr_hbm.at[pl.ds(n0, npw), pl.ds(c * _BC, _BC)], addr_v)
            lax.fori_loop(0, npw, row_body, 0)
            pltpu.sync_copy(
                out_v, out_hbm.at[pl.ds(n0, npw), pl.ds(c * _BC, _BC)])

    return k(addrt, packed.reshape(-1))


def kernel(input, conn_in, conn_state, conn_out, mem_in, mem_state, mem_out):
    pad = ((0, 16 - NB), (0, 0))
    connt_in = jnp.pad(conn_in.T, pad)
    connt_state = jnp.pad(conn_state.T, pad)
    connt_out = jnp.pad(conn_out.T, pad)

    w1 = _build_w(connt_in, INPUT_BITS, 0)
    w2 = _build_w(connt_state, N_IN, 0)
    w3a = _build_w(connt_out, N_IN, 0)
    w3b = _build_w(connt_out, N_STATE, N_IN)

    p1 = _pack(mem_in)
    p2 = _pack(mem_state)
    p3 = _pack(mem_out)

    x0t = input.astype(jnp.float32).T

    addr1t = _mm(w1, x0t)
    b1t = _sc_gather(addr1t, p1)
    addr2t = _mm(w2, b1t)
    b2t = _sc_gather(addr2t, p2)
    addr3t = _mm2(w3a, b1t, w3b, b2t)
    o3t = _sc_gather(addr3t, p3)

    return o3t.T.astype(bool)


# speedup vs baseline: 5.9003x; 1.8289x over previous
import functools

import jax
import jax.numpy as jnp
from jax import lax
from jax.experimental import pallas as pl
from jax.experimental.pallas import tpu as pltpu
from jax.experimental.pallas import tpu_sc as plsc

B = 4096
INPUT_BITS = 1024
N_IN = 2048
N_STATE = 2048
N_OUT = 1024
NB = 12
ADDR = 1 << NB

NC = 2
NS = 16
NW = NC * NS


def _build_w_body(t_off, connt_ref, wlo_ref, whi_ref):
    tb, nb = wlo_ref.shape
    i = pl.program_id(0)
    t_iota = lax.broadcasted_iota(jnp.int32, (tb, nb), 0) + (i * tb + t_off)
    lo = jnp.zeros((tb, nb), jnp.float32)
    hi = jnp.zeros((tb, nb), jnp.float32)
    for k in range(NB):
        row = connt_ref[k, :][None, :]
        hit = row == t_iota
        if k < 8:
            lo = lo + jnp.where(hit, jnp.float32(1 << k), 0.0)
        else:
            hi = hi + jnp.where(hit, jnp.float32(1 << (k - 8)), 0.0)
    wlo_ref[...] = lo.astype(jnp.bfloat16)
    whi_ref[...] = hi.astype(jnp.bfloat16)


def _build_w(connt, t_size, t_off):
    n = connt.shape[1]
    tb = min(512, t_size)
    nb = min(512, n)
    return pl.pallas_call(
        functools.partial(_build_w_body, t_off),
        grid=(t_size // tb, n // nb),
        in_specs=[pl.BlockSpec((16, nb), lambda i, j: (0, j))],
        out_specs=[pl.BlockSpec((tb, nb), lambda i, j: (i, j)),
                   pl.BlockSpec((tb, nb), lambda i, j: (i, j))],
        out_shape=[jax.ShapeDtypeStruct((t_size, n), jnp.bfloat16),
                   jax.ShapeDtypeStruct((t_size, n), jnp.bfloat16)],
    )(connt)


def _pack_body(mem_ref, p_ref):
    jrows, ab = p_ref.shape
    n = mem_ref.shape[0]
    j_iota = lax.broadcasted_iota(jnp.int32, (jrows, n), 0)
    n_iota = lax.broadcasted_iota(jnp.int32, (jrows, n), 1)
    inrow = (n_iota >> 5) == j_iota
    sh = n_iota & 31
    pw_lo = jnp.where(inrow & (sh < 16), (1 << (sh & 15)), 0).astype(jnp.bfloat16)
    pw_hi = jnp.where(inrow & (sh >= 16), (1 << (sh & 15)), 0).astype(jnp.bfloat16)
    memf = mem_ref[...].astype(jnp.bfloat16)
    dn = (((1,), (0,)), ((), ()))
    lo = lax.dot_general(pw_lo, memf, dn, preferred_element_type=jnp.float32)
    hi = lax.dot_general(pw_hi, memf, dn, preferred_element_type=jnp.float32)
    lo_i = (lo + 0.5).astype(jnp.int32)
    hi_i = (hi + 0.5).astype(jnp.int32)
    p_ref[...] = lo_i | (hi_i << 16)


def _pack(mem):
    n, a = mem.shape
    jrows = n // 32
    ab = 512
    return pl.pallas_call(
        _pack_body,
        grid=(a // ab,),
        in_specs=[pl.BlockSpec((n, ab), lambda j: (0, j))],
        out_specs=pl.BlockSpec((jrows, ab), lambda j: (0, j)),
        out_shape=jax.ShapeDtypeStruct((jrows, a), jnp.int32),
    )(mem)


_DN_C00 = (((0,), (0,)), ((), ()))


def _addr_finish(acc_lo, acc_hi, jrows):
    n, bb = acc_lo.shape
    addr = (acc_lo + 256.0 * acc_hi + 0.5).astype(jnp.int32)
    if jrows > 1:
        n_iota = lax.broadcasted_iota(jnp.int32, (n, bb), 0)
        addr = addr + ((n_iota >> 5) & (jrows - 1)) * ADDR
    return addr


def _mm_body(jrows, wlo_ref, whi_ref, x_ref, o_ref):
    xb = x_ref[...].astype(jnp.bfloat16)
    lo = lax.dot_general(wlo_ref[...], xb, _DN_C00,
                         preferred_element_type=jnp.float32)
    hi = lax.dot_general(whi_ref[...], xb, _DN_C00,
                         preferred_element_type=jnp.float32)
    o_ref[...] = _addr_finish(lo, hi, jrows)


def _mm(wlo, whi, x, bb=512):
    t, n = wlo.shape
    b = x.shape[1]
    jrows = (n // NW) // 32
    return pl.pallas_call(
        functools.partial(_mm_body, jrows),
        grid=(b // bb,),
        in_specs=[pl.BlockSpec((t, n), lambda j: (0, 0)),
                  pl.BlockSpec((t, n), lambda j: (0, 0)),
                  pl.BlockSpec((t, bb), lambda j: (0, j))],
        out_specs=pl.BlockSpec((n, bb), lambda j: (0, j)),
        out_shape=jax.ShapeDtypeStruct((n, b), jnp.int32),
    )(wlo, whi, x)


def _mm2_body(jrows, walo_ref, wahi_ref, xa_ref, wblo_ref, wbhi_ref, xb_ref,
              o_ref):
    xa = xa_ref[...].astype(jnp.bfloat16)
    xb = xb_ref[...].astype(jnp.bfloat16)
    lo = lax.dot_general(walo_ref[...], xa, _DN_C00,
                         preferred_element_type=jnp.float32)
    lo = lo + lax.dot_general(wblo_ref[...], xb, _DN_C00,
                              preferred_element_type=jnp.float32)
    hi = lax.dot_general(wahi_ref[...], xa, _DN_C00,
                         preferred_element_type=jnp.float32)
    hi = hi + lax.dot_general(wbhi_ref[...], xb, _DN_C00,
                              preferred_element_type=jnp.float32)
    o_ref[...] = _addr_finish(lo, hi, jrows)


def _mm2(walo, wahi, xa, wblo, wbhi, xb, bb=512):
    t, n = walo.shape
    b = xa.shape[1]
    jrows = (n // NW) // 32
    return pl.pallas_call(
        functools.partial(_mm2_body, jrows),
        grid=(b // bb,),
        in_specs=[pl.BlockSpec((t, n), lambda j: (0, 0)),
                  pl.BlockSpec((t, n), lambda j: (0, 0)),
                  pl.BlockSpec((t, bb), lambda j: (0, j)),
                  pl.BlockSpec((t, n), lambda j: (0, 0)),
                  pl.BlockSpec((t, n), lambda j: (0, 0)),
                  pl.BlockSpec((t, bb), lambda j: (0, j))],
        out_specs=pl.BlockSpec((n, bb), lambda j: (0, j)),
        out_shape=jax.ShapeDtypeStruct((n, b), jnp.int32),
    )(walo, wahi, xa, wblo, wbhi, xb)


_BC = 256


def _sc_gather(addrt, packed):
    n, b = addrt.shape
    npw = n // NW
    jrows = npw // 32
    nchunks = b // _BC
    mesh = plsc.VectorSubcoreMesh(core_axis_name="c", subcore_axis_name="s",
                                  num_cores=NC)

    @functools.partial(
        pl.kernel,
        mesh=mesh,
        compiler_params=pltpu.CompilerParams(needs_layout_passes=False),
        out_type=jax.ShapeDtypeStruct((n, b), jnp.int32),
        scratch_types=[
            pltpu.VMEM((jrows * ADDR,), jnp.int32),
            pltpu.VMEM((npw, _BC), jnp.int32),
            pltpu.VMEM((npw, _BC), jnp.int32),
            pltpu.VMEM((npw, _BC), jnp.int32),
            pltpu.VMEM((npw, _BC), jnp.int32),
            pltpu.SemaphoreType.DMA,
            pltpu.SemaphoreType.DMA,
            pltpu.SemaphoreType.DMA,
            pltpu.SemaphoreType.DMA,
        ],
    )
    def k(addr_hbm, p_hbm, out_hbm, table_v, a0, a1, o0, o1,
          si0, si1, so0, so1):
        wid = lax.axis_index("s") * NC + lax.axis_index("c")
        n0 = wid * npw
        pltpu.sync_copy(p_hbm.at[pl.ds(wid * jrows * ADDR, jrows * ADDR)],
                        table_v)
        abufs, obufs = (a0, a1), (o0, o1)
        isems, osems = (si0, si1), (so0, so1)

        def start_in(c):
            return pltpu.async_copy(
                addr_hbm.at[pl.ds(n0, npw), pl.ds(c * _BC, _BC)],
                abufs[c % 2], isems[c % 2])

        def make_row_body(abuf, obuf):
            def row_body(j, carry):
                sh = jnp.broadcast_to(j & 31, (16,)).astype(jnp.int32)
                for i in range(_BC // 16):
                    a = abuf[j, pl.ds(i * 16, 16)]
                    w16 = plsc.load_gather(table_v, [a])
                    obuf[j, pl.ds(i * 16, 16)] = (
                        lax.shift_right_logical(w16, sh) & 1)
                return carry
            return row_body

        in_h = {0: start_in(0)}
        out_h = {}
        for c in range(nchunks):
            if c + 1 < nchunks:
                in_h[c + 1] = start_in(c + 1)
            in_h[c].wait()
            if c >= 2:
                out_h[c - 2].wait()
            lax.fori_loop(0, npw, make_row_body(abufs[c % 2], obufs[c % 2]), 0)
            out_h[c] = pltpu.async_copy(
                obufs[c % 2],
                out_hbm.at[pl.ds(n0, npw), pl.ds(c * _BC, _BC)],
                osems[c % 2])
        out_h[nchunks - 2].wait()
        out_h[nchunks - 1].wait()

    return k(addrt, packed.reshape(-1))


def kernel(input, conn_in, conn_state, conn_out, mem_in, mem_state, mem_out):
    pad = ((0, 16 - NB), (0, 0))
    connt_in = jnp.pad(conn_in.T, pad)
    connt_state = jnp.pad(conn_state.T, pad)
    connt_out = jnp.pad(conn_out.T, pad)

    w1lo, w1hi = _build_w(connt_in, INPUT_BITS, 0)
    w2lo, w2hi = _build_w(connt_state, N_IN, 0)
    w3alo, w3ahi = _build_w(connt_out, N_IN, 0)
    w3blo, w3bhi = _build_w(connt_out, N_STATE, N_IN)

    p1 = _pack(mem_in)
    p2 = _pack(mem_state)
    p3 = _pack(mem_out)

    x0t = input.astype(jnp.float32).T

    addr1t = _mm(w1lo, w1hi, x0t)
    b1t = _sc_gather(addr1t, p1)
    addr2t = _mm(w2lo, w2hi, b1t)
    b2t = _sc_gather(addr2t, p2)
    addr3t = _mm2(w3alo, w3ahi, b1t, w3blo, w3bhi, b2t)
    o3t = _sc_gather(addr3t, p3)

    return o3t.T.astype(jnp.bool_)


# speedup vs baseline: 6.5552x; 1.1110x over previous
import functools

import jax
import jax.numpy as jnp
from jax import lax
from jax.experimental import pallas as pl
from jax.experimental.pallas import tpu as pltpu
from jax.experimental.pallas import tpu_sc as plsc

B = 4096
INPUT_BITS = 1024
N_IN = 2048
N_STATE = 2048
N_OUT = 1024
NB = 12
ADDR = 1 << NB

NC = 2
NS = 16
NW = NC * NS


def _build_w_body(t_off, connt_ref, wlo_ref, whi_ref):
    tb, nb = wlo_ref.shape
    i = pl.program_id(0)
    t_iota = lax.broadcasted_iota(jnp.int32, (tb, nb), 0) + (i * tb + t_off)
    lo = jnp.zeros((tb, nb), jnp.float32)
    hi = jnp.zeros((tb, nb), jnp.float32)
    for k in range(NB):
        row = connt_ref[k, :][None, :]
        hit = row == t_iota
        if k < 8:
            lo = lo + jnp.where(hit, jnp.float32(1 << k), 0.0)
        else:
            hi = hi + jnp.where(hit, jnp.float32(1 << (k - 8)), 0.0)
    wlo_ref[...] = lo.astype(jnp.bfloat16)
    whi_ref[...] = hi.astype(jnp.bfloat16)


def _build_w(connt, t_size, t_off):
    n = connt.shape[1]
    tb = min(512, t_size)
    nb = min(512, n)
    return pl.pallas_call(
        functools.partial(_build_w_body, t_off),
        grid=(t_size // tb, n // nb),
        in_specs=[pl.BlockSpec((16, nb), lambda i, j: (0, j))],
        out_specs=[pl.BlockSpec((tb, nb), lambda i, j: (i, j)),
                   pl.BlockSpec((tb, nb), lambda i, j: (i, j))],
        out_shape=[jax.ShapeDtypeStruct((t_size, n), jnp.bfloat16),
                   jax.ShapeDtypeStruct((t_size, n), jnp.bfloat16)],
    )(connt)


def _pack_body(mem_ref, p_ref):
    jrows, ab = p_ref.shape
    n = mem_ref.shape[0]
    j_iota = lax.broadcasted_iota(jnp.int32, (jrows, n), 0)
    n_iota = lax.broadcasted_iota(jnp.int32, (jrows, n), 1)
    inrow = (n_iota >> 5) == j_iota
    sh = n_iota & 31
    pw_lo = jnp.where(inrow & (sh < 16), (1 << (sh & 15)), 0).astype(jnp.bfloat16)
    pw_hi = jnp.where(inrow & (sh >= 16), (1 << (sh & 15)), 0).astype(jnp.bfloat16)
    memf = mem_ref[...].astype(jnp.bfloat16)
    dn = (((1,), (0,)), ((), ()))
    lo = lax.dot_general(pw_lo, memf, dn, preferred_element_type=jnp.float32)
    hi = lax.dot_general(pw_hi, memf, dn, preferred_element_type=jnp.float32)
    lo_i = (lo + 0.5).astype(jnp.int32)
    hi_i = (hi + 0.5).astype(jnp.int32)
    p_ref[...] = lo_i | (hi_i << 16)


def _pack(mem):
    n, a = mem.shape
    jrows = n // 32
    ab = 512
    return pl.pallas_call(
        _pack_body,
        grid=(a // ab,),
        in_specs=[pl.BlockSpec((n, ab), lambda j: (0, j))],
        out_specs=pl.BlockSpec((jrows, ab), lambda j: (0, j)),
        out_shape=jax.ShapeDtypeStruct((jrows, a), jnp.int32),
    )(mem)


_DN_C00 = (((0,), (0,)), ((), ()))


def _to_bits_bf16(x, raw):
    if raw:
        t, bb = x.shape
        sh = lax.broadcasted_iota(jnp.int32, (t, bb), 0) & 31
        x = lax.shift_right_logical(x, sh) & 1
    return x.astype(jnp.bfloat16)


def _addr_finish(acc_lo, acc_hi, jrows):
    n, bb = acc_lo.shape
    addr = (acc_lo + 256.0 * acc_hi + 0.5).astype(jnp.int32)
    if jrows > 1:
        n_iota = lax.broadcasted_iota(jnp.int32, (n, bb), 0)
        addr = addr + ((n_iota >> 5) & (jrows - 1)) * ADDR
    return addr


def _mm_body(jrows, raw, wlo_ref, whi_ref, x_ref, o_ref):
    xb = _to_bits_bf16(x_ref[...], raw)
    lo = lax.dot_general(wlo_ref[...], xb, _DN_C00,
                         preferred_element_type=jnp.float32)
    hi = lax.dot_general(whi_ref[...], xb, _DN_C00,
                         preferred_element_type=jnp.float32)
    o_ref[...] = _addr_finish(lo, hi, jrows)


def _mm(wlo, whi, x, raw, bb=512):
    t, n = wlo.shape
    b = x.shape[1]
    jrows = (n // NW) // 32
    return pl.pallas_call(
        functools.partial(_mm_body, jrows, raw),
        grid=(b // bb,),
        in_specs=[pl.BlockSpec((t, n), lambda j: (0, 0)),
                  pl.BlockSpec((t, n), lambda j: (0, 0)),
                  pl.BlockSpec((t, bb), lambda j: (0, j))],
        out_specs=pl.BlockSpec((n, bb), lambda j: (0, j)),
        out_shape=jax.ShapeDtypeStruct((n, b), jnp.int32),
    )(wlo, whi, x)


def _mm2_body(jrows, walo_ref, wahi_ref, xa_ref, wblo_ref, wbhi_ref, xb_ref,
              o_ref):
    xa = _to_bits_bf16(xa_ref[...], True)
    xb = _to_bits_bf16(xb_ref[...], True)
    lo = lax.dot_general(walo_ref[...], xa, _DN_C00,
                         preferred_element_type=jnp.float32)
    lo = lo + lax.dot_general(wblo_ref[...], xb, _DN_C00,
                              preferred_element_type=jnp.float32)
    hi = lax.dot_general(wahi_ref[...], xa, _DN_C00,
                         preferred_element_type=jnp.float32)
    hi = hi + lax.dot_general(wbhi_ref[...], xb, _DN_C00,
                              preferred_element_type=jnp.float32)
    o_ref[...] = _addr_finish(lo, hi, jrows)


def _mm2(walo, wahi, xa, wblo, wbhi, xb, bb=512):
    t, n = walo.shape
    b = xa.shape[1]
    jrows = (n // NW) // 32
    return pl.pallas_call(
        functools.partial(_mm2_body, jrows),
        grid=(b // bb,),
        in_specs=[pl.BlockSpec((t, n), lambda j: (0, 0)),
                  pl.BlockSpec((t, n), lambda j: (0, 0)),
                  pl.BlockSpec((t, bb), lambda j: (0, j)),
                  pl.BlockSpec((t, n), lambda j: (0, 0)),
                  pl.BlockSpec((t, n), lambda j: (0, 0)),
                  pl.BlockSpec((t, bb), lambda j: (0, j))],
        out_specs=pl.BlockSpec((n, bb), lambda j: (0, j)),
        out_shape=jax.ShapeDtypeStruct((n, b), jnp.int32),
    )(walo, wahi, xa, wblo, wbhi, xb)


_BC = 256


def _sc_gather(addrt, packed, extract):
    n, b = addrt.shape
    npw = n // NW
    jrows = npw // 32
    nchunks = b // _BC
    mesh = plsc.VectorSubcoreMesh(core_axis_name="c", subcore_axis_name="s",
                                  num_cores=NC)

    @functools.partial(
        pl.kernel,
        mesh=mesh,
        compiler_params=pltpu.CompilerParams(needs_layout_passes=False),
        out_type=jax.ShapeDtypeStruct((n, b), jnp.int32),
        scratch_types=[
            pltpu.VMEM((jrows * ADDR,), jnp.int32),
            pltpu.VMEM((npw, _BC), jnp.int32),
            pltpu.VMEM((npw, _BC), jnp.int32),
            pltpu.VMEM((npw, _BC), jnp.int32),
            pltpu.VMEM((npw, _BC), jnp.int32),
            pltpu.SemaphoreType.DMA,
            pltpu.SemaphoreType.DMA,
            pltpu.SemaphoreType.DMA,
            pltpu.SemaphoreType.DMA,
        ],
    )
    def k(addr_hbm, p_hbm, out_hbm, table_v, a0, a1, o0, o1,
          si0, si1, so0, so1):
        wid = lax.axis_index("s") * NC + lax.axis_index("c")
        n0 = wid * npw
        pltpu.sync_copy(p_hbm.at[pl.ds(wid * jrows * ADDR, jrows * ADDR)],
                        table_v)
        abufs, obufs = (a0, a1), (o0, o1)
        isems, osems = (si0, si1), (so0, so1)

        def start_in(c):
            return pltpu.async_copy(
                addr_hbm.at[pl.ds(n0, npw), pl.ds(c * _BC, _BC)],
                abufs[c % 2], isems[c % 2])

        def run_rows(abuf, obuf):
            def row_body(j, carry):
                sh = jnp.broadcast_to(j & 31, (16,)).astype(jnp.int32)
                for i in range(_BC // 16):
                    a = abuf[j, pl.ds(i * 16, 16)]
                    w16 = plsc.load_gather(table_v, [a])
                    if extract:
                        w16 = lax.shift_right_logical(w16, sh) & 1
                    obuf[j, pl.ds(i * 16, 16)] = w16
                return carry
            lax.fori_loop(0, npw, row_body, 0)

        in_h = {0: start_in(0)}
        out_h = {}
        for c in range(nchunks):
            if c + 1 < nchunks:
                in_h[c + 1] = start_in(c + 1)
            in_h[c].wait()
            if c >= 2:
                out_h[c - 2].wait()
            run_rows(abufs[c % 2], obufs[c % 2])
            out_h[c] = pltpu.async_copy(
                obufs[c % 2],
                out_hbm.at[pl.ds(n0, npw), pl.ds(c * _BC, _BC)],
                osems[c % 2])
        out_h[nchunks - 2].wait()
        out_h[nchunks - 1].wait()

    return k(addrt, packed.reshape(-1))


def kernel(input, conn_in, conn_state, conn_out, mem_in, mem_state, mem_out):
    pad = ((0, 16 - NB), (0, 0))
    connt_in = jnp.pad(conn_in.T, pad)
    connt_state = jnp.pad(conn_state.T, pad)
    connt_out = jnp.pad(conn_out.T, pad)

    x0t = input.astype(jnp.float32).T
    w1lo, w1hi = _build_w(connt_in, INPUT_BITS, 0)
    p1 = _pack(mem_in)
    addr1t = _mm(w1lo, w1hi, x0t, raw=False)
    v1t = _sc_gather(addr1t, p1, extract=False)

    w2lo, w2hi = _build_w(connt_state, N_IN, 0)
    p2 = _pack(mem_state)
    addr2t = _mm(w2lo, w2hi, v1t, raw=True)
    v2t = _sc_gather(addr2t, p2, extract=False)

    w3alo, w3ahi = _build_w(connt_out, N_IN, 0)
    w3blo, w3bhi = _build_w(connt_out, N_STATE, N_IN)
    p3 = _pack(mem_out)
    addr3t = _mm2(w3alo, w3ahi, v1t, w3blo, w3bhi, v2t)
    o3t = _sc_gather(addr3t, p3, extract=True)

    return o3t.T.astype(jnp.bool_)


# speedup vs baseline: 6.7978x; 1.0370x over previous
import functools

import jax
import jax.numpy as jnp
from jax import lax
from jax.experimental import pallas as pl
from jax.experimental.pallas import tpu as pltpu
from jax.experimental.pallas import tpu_sc as plsc

B = 4096
INPUT_BITS = 1024
N_IN = 2048
N_STATE = 2048
N_OUT = 1024
NB = 12
ADDR = 1 << NB

NC = 2
NS = 16
NW = NC * NS


def _build_w_body(t_off, connt_ref, wlo_ref, whi_ref):
    tb, nb = wlo_ref.shape
    i = pl.program_id(0)
    t_iota = lax.broadcasted_iota(jnp.int32, (tb, nb), 0) + (i * tb + t_off)
    lo = jnp.zeros((tb, nb), jnp.int32)
    hi = jnp.zeros((tb, nb), jnp.int32)
    for k in range(NB):
        row = connt_ref[k, :][None, :]
        hit = row == t_iota
        if k < 6:
            lo = lo + jnp.where(hit, jnp.int32(1 << k), 0)
        else:
            hi = hi + jnp.where(hit, jnp.int32(1 << (k - 6)), 0)
    wlo_ref[...] = lo.astype(jnp.int8)
    whi_ref[...] = hi.astype(jnp.int8)


def _build_w(connt, t_size, t_off):
    n = connt.shape[1]
    tb = min(512, t_size)
    nb = min(512, n)
    return pl.pallas_call(
        functools.partial(_build_w_body, t_off),
        grid=(t_size // tb, n // nb),
        in_specs=[pl.BlockSpec((16, nb), lambda i, j: (0, j))],
        out_specs=[pl.BlockSpec((tb, nb), lambda i, j: (i, j)),
                   pl.BlockSpec((tb, nb), lambda i, j: (i, j))],
        out_shape=[jax.ShapeDtypeStruct((t_size, n), jnp.int8),
                   jax.ShapeDtypeStruct((t_size, n), jnp.int8)],
    )(connt)


def _pack_body(mem_ref, p_ref):
    jrows, ab = p_ref.shape
    n = mem_ref.shape[0]
    j_iota = lax.broadcasted_iota(jnp.int32, (jrows, n), 0)
    n_iota = lax.broadcasted_iota(jnp.int32, (jrows, n), 1)
    inrow = (n_iota >> 5) == j_iota
    sh = n_iota & 31
    pw_lo = jnp.where(inrow & (sh < 16), (1 << (sh & 15)), 0).astype(jnp.bfloat16)
    pw_hi = jnp.where(inrow & (sh >= 16), (1 << (sh & 15)), 0).astype(jnp.bfloat16)
    memf = mem_ref[...].astype(jnp.bfloat16)
    dn = (((1,), (0,)), ((), ()))
    lo = lax.dot_general(pw_lo, memf, dn, preferred_element_type=jnp.float32)
    hi = lax.dot_general(pw_hi, memf, dn, preferred_element_type=jnp.float32)
    lo_i = (lo + 0.5).astype(jnp.int32)
    hi_i = (hi + 0.5).astype(jnp.int32)
    p_ref[...] = lo_i | (hi_i << 16)


def _pack(mem):
    n, a = mem.shape
    jrows = n // 32
    ab = 512
    return pl.pallas_call(
        _pack_body,
        grid=(a // ab,),
        in_specs=[pl.BlockSpec((n, ab), lambda j: (0, j))],
        out_specs=pl.BlockSpec((jrows, ab), lambda j: (0, j)),
        out_shape=jax.ShapeDtypeStruct((jrows, a), jnp.int32),
    )(mem)


_DN_C00 = (((0,), (0,)), ((), ()))


def _to_bits_i8(x, raw):
    if raw:
        t, bb = x.shape
        sh = lax.broadcasted_iota(jnp.int32, (t, bb), 0) & 31
        x = lax.shift_right_logical(x, sh) & 1
    return x.astype(jnp.int8)


def _addr_finish(acc_lo, acc_hi, jrows):
    n, bb = acc_lo.shape
    addr = acc_lo + (acc_hi << 6)
    if jrows > 1:
        n_iota = lax.broadcasted_iota(jnp.int32, (n, bb), 0)
        addr = addr + ((n_iota >> 5) & (jrows - 1)) * ADDR
    return addr


def _mm_body(jrows, raw, wlo_ref, whi_ref, x_ref, o_ref):
    xb = _to_bits_i8(x_ref[...], raw)
    lo = lax.dot_general(wlo_ref[...], xb, _DN_C00,
                         preferred_element_type=jnp.int32)
    hi = lax.dot_general(whi_ref[...], xb, _DN_C00,
                         preferred_element_type=jnp.int32)
    o_ref[...] = _addr_finish(lo, hi, jrows)


def _mm(wlo, whi, x, raw, bb=512):
    t, n = wlo.shape
    b = x.shape[1]
    jrows = (n // NW) // 32
    return pl.pallas_call(
        functools.partial(_mm_body, jrows, raw),
        grid=(b // bb,),
        in_specs=[pl.BlockSpec((t, n), lambda j: (0, 0)),
                  pl.BlockSpec((t, n), lambda j: (0, 0)),
                  pl.BlockSpec((t, bb), lambda j: (0, j))],
        out_specs=pl.BlockSpec((n, bb), lambda j: (0, j)),
        out_shape=jax.ShapeDtypeStruct((n, b), jnp.int32),
    )(wlo, whi, x)


def _mm2_body(jrows, walo_ref, wahi_ref, xa_ref, wblo_ref, wbhi_ref, xb_ref,
              o_ref):
    xa = _to_bits_i8(xa_ref[...], True)
    xb = _to_bits_i8(xb_ref[...], True)
    lo = lax.dot_general(walo_ref[...], xa, _DN_C00,
                         preferred_element_type=jnp.int32)
    lo = lo + lax.dot_general(wblo_ref[...], xb, _DN_C00,
                              preferred_element_type=jnp.int32)
    hi = lax.dot_general(wahi_ref[...], xa, _DN_C00,
                         preferred_element_type=jnp.int32)
    hi = hi + lax.dot_general(wbhi_ref[...], xb, _DN_C00,
                              preferred_element_type=jnp.int32)
    o_ref[...] = _addr_finish(lo, hi, jrows)


def _mm2(walo, wahi, xa, wblo, wbhi, xb, bb=512):
    t, n = walo.shape
    b = xa.shape[1]
    jrows = (n // NW) // 32
    return pl.pallas_call(
        functools.partial(_mm2_body, jrows),
        grid=(b // bb,),
        in_specs=[pl.BlockSpec((t, n), lambda j: (0, 0)),
                  pl.BlockSpec((t, n), lambda j: (0, 0)),
                  pl.BlockSpec((t, bb), lambda j: (0, j)),
                  pl.BlockSpec((t, n), lambda j: (0, 0)),
                  pl.BlockSpec((t, n), lambda j: (0, 0)),
                  pl.BlockSpec((t, bb), lambda j: (0, j))],
        out_specs=pl.BlockSpec((n, bb), lambda j: (0, j)),
        out_shape=jax.ShapeDtypeStruct((n, b), jnp.int32),
    )(walo, wahi, xa, wblo, wbhi, xb)


_BC = 256


def _sc_gather(addrt, packed, extract):
    n, b = addrt.shape
    npw = n // NW
    jrows = npw // 32
    nchunks = b // _BC
    mesh = plsc.VectorSubcoreMesh(core_axis_name="c", subcore_axis_name="s",
                                  num_cores=NC)

    @functools.partial(
        pl.kernel,
        mesh=mesh,
        compiler_params=pltpu.CompilerParams(needs_layout_passes=False),
        out_type=jax.ShapeDtypeStruct((n, b), jnp.int32),
        scratch_types=[
            pltpu.VMEM((jrows * ADDR,), jnp.int32),
            pltpu.VMEM((npw, _BC), jnp.int32),
            pltpu.VMEM((npw, _BC), jnp.int32),
            pltpu.VMEM((npw, _BC), jnp.int32),
            pltpu.VMEM((npw, _BC), jnp.int32),
            pltpu.SemaphoreType.DMA,
            pltpu.SemaphoreType.DMA,
            pltpu.SemaphoreType.DMA,
            pltpu.SemaphoreType.DMA,
        ],
    )
    def k(addr_hbm, p_hbm, out_hbm, table_v, a0, a1, o0, o1,
          si0, si1, so0, so1):
        wid = lax.axis_index("s") * NC + lax.axis_index("c")
        n0 = wid * npw
        pltpu.sync_copy(p_hbm.at[pl.ds(wid * jrows * ADDR, jrows * ADDR)],
                        table_v)
        abufs, obufs = (a0, a1), (o0, o1)
        isems, osems = (si0, si1), (so0, so1)

        def start_in(c):
            return pltpu.async_copy(
                addr_hbm.at[pl.ds(n0, npw), pl.ds(c * _BC, _BC)],
                abufs[c % 2], isems[c % 2])

        def run_rows(abuf, obuf):
            def row_body(j, carry):
                sh = jnp.broadcast_to(j & 31, (16,)).astype(jnp.int32)
                for i in range(_BC // 16):
                    a = abuf[j, pl.ds(i * 16, 16)]
                    w16 = plsc.load_gather(table_v, [a])
                    if extract:
                        w16 = lax.shift_right_logical(w16, sh) & 1
                    obuf[j, pl.ds(i * 16, 16)] = w16
                return carry
            lax.fori_loop(0, npw, row_body, 0)

        in_h = {0: start_in(0)}
        out_h = {}
        for c in range(nchunks):
            if c + 1 < nchunks:
                in_h[c + 1] = start_in(c + 1)
            in_h[c].wait()
            if c >= 2:
                out_h[c - 2].wait()
            run_rows(abufs[c % 2], obufs[c % 2])
            out_h[c] = pltpu.async_copy(
                obufs[c % 2],
                out_hbm.at[pl.ds(n0, npw), pl.ds(c * _BC, _BC)],
                osems[c % 2])
        out_h[nchunks - 2].wait()
        out_h[nchunks - 1].wait()

    return k(addrt, packed.reshape(-1))


def kernel(input, conn_in, conn_state, conn_out, mem_in, mem_state, mem_out):
    pad = ((0, 16 - NB), (0, 0))
    connt_in = jnp.pad(conn_in.T, pad)
    connt_state = jnp.pad(conn_state.T, pad)
    connt_out = jnp.pad(conn_out.T, pad)

    x0t = input.T.astype(jnp.int8)
    w1lo, w1hi = _build_w(connt_in, INPUT_BITS, 0)
    p1 = _pack(mem_in)
    addr1t = _mm(w1lo, w1hi, x0t, raw=False)
    v1t = _sc_gather(addr1t, p1, extract=False)

    w2lo, w2hi = _build_w(connt_state, N_IN, 0)
    p2 = _pack(mem_state)
    addr2t = _mm(w2lo, w2hi, v1t, raw=True)
    v2t = _sc_gather(addr2t, p2, extract=False)

    w3alo, w3ahi = _build_w(connt_out, N_IN, 0)
    w3blo, w3bhi = _build_w(connt_out, N_STATE, N_IN)
    p3 = _pack(mem_out)
    addr3t = _mm2(w3alo, w3ahi, v1t, w3blo, w3bhi, v2t)
    o3t = _sc_gather(addr3t, p3, extract=True)

    return o3t.T.astype(jnp.bool_)


# speedup vs baseline: 7.8376x; 1.1529x over previous
import functools

import jax
import jax.numpy as jnp
from jax import lax
from jax.experimental import pallas as pl
from jax.experimental.pallas import tpu as pltpu
from jax.experimental.pallas import tpu_sc as plsc

B = 4096
INPUT_BITS = 1024
N_IN = 2048
N_STATE = 2048
N_OUT = 1024
NB = 12
ADDR = 1 << NB

NC = 2
NS = 16
NW = NC * NS


def _build_w_body(t_off, connt_ref, wlo_ref, whi_ref):
    tb, nb = wlo_ref.shape
    i = pl.program_id(0)
    t_iota = lax.broadcasted_iota(jnp.int32, (tb, nb), 0) + (i * tb + t_off)
    lo = jnp.zeros((tb, nb), jnp.int32)
    hi = jnp.zeros((tb, nb), jnp.int32)
    for k in range(NB):
        row = connt_ref[k, :][None, :]
        hit = row == t_iota
        if k < 6:
            lo = lo + jnp.where(hit, jnp.int32(1 << k), 0)
        else:
            hi = hi + jnp.where(hit, jnp.int32(1 << (k - 6)), 0)
    wlo_ref[...] = lo.astype(jnp.int8)
    whi_ref[...] = hi.astype(jnp.int8)


def _build_w(connt, t_size, t_off):
    n = connt.shape[1]
    tb = min(512, t_size)
    nb = min(512, n)
    return pl.pallas_call(
        functools.partial(_build_w_body, t_off),
        grid=(t_size // tb, n // nb),
        in_specs=[pl.BlockSpec((16, nb), lambda i, j: (0, j))],
        out_specs=[pl.BlockSpec((tb, nb), lambda i, j: (i, j)),
                   pl.BlockSpec((tb, nb), lambda i, j: (i, j))],
        out_shape=[jax.ShapeDtypeStruct((t_size, n), jnp.int8),
                   jax.ShapeDtypeStruct((t_size, n), jnp.int8)],
    )(connt)


def _pack_body(mem_ref, p_ref):
    jrows, ab = p_ref.shape
    n = mem_ref.shape[0]
    j_iota = lax.broadcasted_iota(jnp.int32, (jrows, n), 0)
    n_iota = lax.broadcasted_iota(jnp.int32, (jrows, n), 1)
    inrow = (n_iota >> 5) == j_iota
    sh = n_iota & 31
    pw_lo = jnp.where(inrow & (sh < 16), (1 << (sh & 15)), 0).astype(jnp.bfloat16)
    pw_hi = jnp.where(inrow & (sh >= 16), (1 << (sh & 15)), 0).astype(jnp.bfloat16)
    memf = mem_ref[...].astype(jnp.bfloat16)
    dn = (((1,), (0,)), ((), ()))
    lo = lax.dot_general(pw_lo, memf, dn, preferred_element_type=jnp.float32)
    hi = lax.dot_general(pw_hi, memf, dn, preferred_element_type=jnp.float32)
    lo_i = (lo + 0.5).astype(jnp.int32)
    hi_i = (hi + 0.5).astype(jnp.int32)
    p_ref[...] = lo_i | (hi_i << 16)


def _pack(mem):
    n, a = mem.shape
    jrows = n // 32
    ab = 512
    return pl.pallas_call(
        _pack_body,
        grid=(a // ab,),
        in_specs=[pl.BlockSpec((n, ab), lambda j: (0, j))],
        out_specs=pl.BlockSpec((jrows, ab), lambda j: (0, j)),
        out_shape=jax.ShapeDtypeStruct((jrows, a), jnp.int32),
    )(mem)


_DN_C00 = (((0,), (0,)), ((), ()))


def _to_bits_i8(x, raw):
    if raw:
        t, bb = x.shape
        sh = lax.broadcasted_iota(jnp.int32, (t, bb), 0) & 31
        x = lax.shift_right_logical(x, sh) & 1
    return x.astype(jnp.int8)


def _addr_finish(acc_lo, acc_hi, jrows):
    n, bb = acc_lo.shape
    addr = acc_lo + (acc_hi << 6)
    if jrows > 1:
        n_iota = lax.broadcasted_iota(jnp.int32, (n, bb), 0)
        addr = addr + ((n_iota >> 5) & (jrows - 1)) * ADDR
    return addr


def _mm_body(jrows, raw, wlo_ref, whi_ref, x_ref, o_ref):
    xb = _to_bits_i8(x_ref[...], raw)
    lo = lax.dot_general(wlo_ref[...], xb, _DN_C00,
                         preferred_element_type=jnp.int32)
    hi = lax.dot_general(whi_ref[...], xb, _DN_C00,
                         preferred_element_type=jnp.int32)
    o_ref[...] = _addr_finish(lo, hi, jrows)


def _mm(wlo, whi, x, raw, bb=512):
    t, n = wlo.shape
    b = x.shape[1]
    jrows = (n // NW) // 32
    return pl.pallas_call(
        functools.partial(_mm_body, jrows, raw),
        grid=(b // bb,),
        in_specs=[pl.BlockSpec((t, n), lambda j: (0, 0)),
                  pl.BlockSpec((t, n), lambda j: (0, 0)),
                  pl.BlockSpec((t, bb), lambda j: (0, j))],
        out_specs=pl.BlockSpec((n, bb), lambda j: (0, j)),
        out_shape=jax.ShapeDtypeStruct((n, b), jnp.int32),
    )(wlo, whi, x)


def _mm2_body(jrows, walo_ref, wahi_ref, xa_ref, wblo_ref, wbhi_ref, xb_ref,
              o_ref):
    xa = _to_bits_i8(xa_ref[...], True)
    xb = _to_bits_i8(xb_ref[...], True)
    lo = lax.dot_general(walo_ref[...], xa, _DN_C00,
                         preferred_element_type=jnp.int32)
    lo = lo + lax.dot_general(wblo_ref[...], xb, _DN_C00,
                              preferred_element_type=jnp.int32)
    hi = lax.dot_general(wahi_ref[...], xa, _DN_C00,
                         preferred_element_type=jnp.int32)
    hi = hi + lax.dot_general(wbhi_ref[...], xb, _DN_C00,
                              preferred_element_type=jnp.int32)
    o_ref[...] = _addr_finish(lo, hi, jrows)


def _mm2(walo, wahi, xa, wblo, wbhi, xb, bb=512):
    t, n = walo.shape
    b = xa.shape[1]
    jrows = (n // NW) // 32
    return pl.pallas_call(
        functools.partial(_mm2_body, jrows),
        grid=(b // bb,),
        in_specs=[pl.BlockSpec((t, n), lambda j: (0, 0)),
                  pl.BlockSpec((t, n), lambda j: (0, 0)),
                  pl.BlockSpec((t, bb), lambda j: (0, j)),
                  pl.BlockSpec((t, n), lambda j: (0, 0)),
                  pl.BlockSpec((t, n), lambda j: (0, 0)),
                  pl.BlockSpec((t, bb), lambda j: (0, j))],
        out_specs=pl.BlockSpec((n, bb), lambda j: (0, j)),
        out_shape=jax.ShapeDtypeStruct((n, b), jnp.int32),
    )(walo, wahi, xa, wblo, wbhi, xb)


_BC = 256


def _sc_gather(addrt, packed, extract):
    n, b = addrt.shape
    npw = n // NW
    jrows = npw // 32
    nchunks = b // _BC
    mesh = plsc.VectorSubcoreMesh(core_axis_name="c", subcore_axis_name="s",
                                  num_cores=NC)

    @functools.partial(
        pl.kernel,
        mesh=mesh,
        compiler_params=pltpu.CompilerParams(needs_layout_passes=False),
        out_type=jax.ShapeDtypeStruct((n, b), jnp.int32),
        scratch_types=[
            pltpu.VMEM((jrows * ADDR,), jnp.int32),
            pltpu.VMEM((npw, _BC), jnp.int32),
            pltpu.VMEM((npw, _BC), jnp.int32),
            pltpu.VMEM((npw, _BC), jnp.int32),
            pltpu.VMEM((npw, _BC), jnp.int32),
            pltpu.SemaphoreType.DMA,
            pltpu.SemaphoreType.DMA,
            pltpu.SemaphoreType.DMA,
            pltpu.SemaphoreType.DMA,
        ],
    )
    def k(addr_hbm, p_hbm, out_hbm, table_v, a0, a1, o0, o1,
          si0, si1, so0, so1):
        wid = lax.axis_index("s") * NC + lax.axis_index("c")
        n0 = wid * npw
        pltpu.sync_copy(p_hbm.at[pl.ds(wid * jrows * ADDR, jrows * ADDR)],
                        table_v)
        abufs, obufs = (a0, a1), (o0, o1)
        isems, osems = (si0, si1), (so0, so1)

        def start_in(c):
            return pltpu.async_copy(
                addr_hbm.at[pl.ds(n0, npw), pl.ds(c * _BC, _BC)],
                abufs[c % 2], isems[c % 2])

        def run_rows(abuf, obuf):
            if extract:
                def row_body(j, carry):
                    sh = jnp.broadcast_to(j & 31, (16,)).astype(jnp.int32)
                    for i in range(_BC // 16):
                        a = abuf[j, pl.ds(i * 16, 16)]
                        w16 = plsc.load_gather(table_v, [a])
                        obuf[j, pl.ds(i * 16, 16)] = (
                            lax.shift_right_logical(w16, sh) & 1)
                    return carry
                lax.fori_loop(0, npw, row_body, 0)
            else:
                @plsc.parallel_loop(0, npw, 1, unroll=4)
                def row_body(j):
                    for i in range(_BC // 16):
                        a = abuf[j, pl.ds(i * 16, 16)]
                        obuf[j, pl.ds(i * 16, 16)] = plsc.load_gather(
                            table_v, [a])

        in_h = {0: start_in(0)}
        out_h = {}
        for c in range(nchunks):
            if c + 1 < nchunks:
                in_h[c + 1] = start_in(c + 1)
            in_h[c].wait()
            if c >= 2:
                out_h[c - 2].wait()
            run_rows(abufs[c % 2], obufs[c % 2])
            out_h[c] = pltpu.async_copy(
                obufs[c % 2],
                out_hbm.at[pl.ds(n0, npw), pl.ds(c * _BC, _BC)],
                osems[c % 2])
        out_h[nchunks - 2].wait()
        out_h[nchunks - 1].wait()

    return k(addrt, packed.reshape(-1))


def kernel(input, conn_in, conn_state, conn_out, mem_in, mem_state, mem_out):
    pad = ((0, 16 - NB), (0, 0))
    connt_in = jnp.pad(conn_in.T, pad)
    connt_state = jnp.pad(conn_state.T, pad)
    connt_out = jnp.pad(conn_out.T, pad)

    x0t = input.T.astype(jnp.int8)
    w1lo, w1hi = _build_w(connt_in, INPUT_BITS, 0)
    p1 = _pack(mem_in)
    addr1t = _mm(w1lo, w1hi, x0t, raw=False)
    v1t = _sc_gather(addr1t, p1, extract=False)

    w2lo, w2hi = _build_w(connt_state, N_IN, 0)
    p2 = _pack(mem_state)
    addr2t = _mm(w2lo, w2hi, v1t, raw=True)
    v2t = _sc_gather(addr2t, p2, extract=False)

    w3alo, w3ahi = _build_w(connt_out, N_IN, 0)
    w3blo, w3bhi = _build_w(connt_out, N_STATE, N_IN)
    p3 = _pack(mem_out)
    addr3t = _mm2(w3alo, w3ahi, v1t, w3blo, w3bhi, v2t)
    o3t = _sc_gather(addr3t, p3, extract=True)

    return o3t.T.astype(jnp.bool_)
